# Initial kernel scaffold; baseline (speedup 1.0000x reference)
import jax
import jax.numpy as jnp
from jax.experimental import pallas as pl


def kernel(x, pattern_i, pattern_q, segment_scale):
    raise NotImplementedError("write your pallas kernel here")


# speedup vs baseline: 11.2537x; 11.2537x over previous
import functools

import jax
import jax.numpy as jnp
import numpy as np
from jax.experimental import pallas as pl
from jax.experimental.pallas import tpu as pltpu

_BASE_AMP = 0.08


def _anchor_starts(signal_len, num_segments, seg_length):
    max_start = max(signal_len - seg_length, 0)
    head = 0.1 * signal_len
    tail = max(0.0, 0.78 * signal_len)
    anchors = np.linspace(head, tail, num_segments)
    return np.clip(np.round(anchors), 0, max_start).astype(np.int64)


def _body(x_ref, pi_ref, pq_ref, sc_ref, o_ref, trig_ref, *,
          starts, seg_len, sig_len):
    S = sig_len
    pat_i = jnp.tanh(pi_ref[...])
    pat_q = jnp.tanh(pq_ref[...])
    trig_ref[...] = jnp.zeros((1, 2 * S), jnp.float32)
    for k, s in enumerate(starts):
        e = min(s + seg_len, S)
        L = e - s
        g = jnp.maximum(sc_ref[k], 0.0)
        for base, pat in ((s, pat_i), (S + s, pat_q)):
            trig_ref[:, pl.ds(base, L)] += g * pat[:, :L]
    xv = x_ref[...]
    ssq = jnp.sum(xv * xv, axis=1, keepdims=True)
    amp = _BASE_AMP * jnp.sqrt(ssq / (2.0 * S) + 1e-12)
    o_ref[...] = xv + amp * trig_ref[...]


def kernel(x, pattern_i, pattern_q, segment_scale):
    batch, ch, S = x.shape
    L = pattern_i.shape[0]
    nseg = segment_scale.shape[0]
    starts = tuple(int(v) for v in _anchor_starts(S, nseg, L))

    x2 = x.reshape(batch, ch * S)
    TB = 8
    body = functools.partial(_body, starts=starts, seg_len=L, sig_len=S)
    out = pl.pallas_call(
        body,
        grid=(batch // TB,),
        in_specs=[
            pl.BlockSpec((TB, ch * S), lambda i: (i, 0)),
            pl.BlockSpec((1, L), lambda i: (0, 0)),
            pl.BlockSpec((1, L), lambda i: (0, 0)),
            pl.BlockSpec(memory_space=pltpu.SMEM),
        ],
        out_specs=pl.BlockSpec((TB, ch * S), lambda i: (i, 0)),
        out_shape=jax.ShapeDtypeStruct((batch, ch * S), jnp.float32),
        scratch_shapes=[pltpu.VMEM((1, ch * S), jnp.float32)],
    )(x2, pattern_i.reshape(1, L), pattern_q.reshape(1, L), segment_scale)
    return out.reshape(batch, ch, S)


# speedup vs baseline: 14.2402x; 1.2654x over previous
import functools

import jax
import jax.numpy as jnp
import numpy as np
from jax.experimental import pallas as pl
from jax.experimental.pallas import tpu as pltpu

_BASE_AMP = 0.08


def _anchor_starts(signal_len, num_segments, seg_length):
    max_start = max(signal_len - seg_length, 0)
    head = 0.1 * signal_len
    tail = max(0.0, 0.78 * signal_len)
    anchors = np.linspace(head, tail, num_segments)
    return np.clip(np.round(anchors), 0, max_start).astype(np.int64)


def _body(x_ref, pi_ref, pq_ref, sc_ref, o_ref, trig_ref, *,
          starts, seg_len, sig_len):
    S = sig_len
    pat_i = jnp.tanh(pi_ref[...])
    pat_q = jnp.tanh(pq_ref[...])
    trig_ref[...] = jnp.zeros((1, 2 * S), jnp.float32)
    for k, s in enumerate(starts):
        e = min(s + seg_len, S)
        L = e - s
        g = jnp.maximum(sc_ref[k], 0.0)
        for base, pat in ((s, pat_i), (S + s, pat_q)):
            trig_ref[:, pl.ds(base, L)] += g * pat[:, :L]
    xv = x_ref[...]
    ssq = jnp.sum(xv * xv, axis=1, keepdims=True)
    amp = _BASE_AMP * jnp.sqrt(ssq / (2.0 * S) + 1e-12)
    o_ref[...] = xv + amp * trig_ref[...]


def kernel(x, pattern_i, pattern_q, segment_scale):
    batch, ch, S = x.shape
    L = pattern_i.shape[0]
    nseg = segment_scale.shape[0]
    starts = tuple(int(v) for v in _anchor_starts(S, nseg, L))

    x2 = x.reshape(batch, ch * S)
    TB = 32
    body = functools.partial(_body, starts=starts, seg_len=L, sig_len=S)
    out = pl.pallas_call(
        body,
        grid=(batch // TB,),
        in_specs=[
            pl.BlockSpec((TB, ch * S), lambda i: (i, 0)),
            pl.BlockSpec((1, L), lambda i: (0, 0)),
            pl.BlockSpec((1, L), lambda i: (0, 0)),
            pl.BlockSpec(memory_space=pltpu.SMEM),
        ],
        out_specs=pl.BlockSpec((TB, ch * S), lambda i: (i, 0)),
        out_shape=jax.ShapeDtypeStruct((batch, ch * S), jnp.float32),
        scratch_shapes=[pltpu.VMEM((1, ch * S), jnp.float32)],
        compiler_params=pltpu.CompilerParams(
            dimension_semantics=("parallel",)),
    )(x2, pattern_i.reshape(1, L), pattern_q.reshape(1, L), segment_scale)
    return out.reshape(batch, ch, S)


# speedup vs baseline: 14.7259x; 1.0341x over previous
import functools

import jax
import jax.numpy as jnp
import numpy as np
from jax.experimental import pallas as pl
from jax.experimental.pallas import tpu as pltpu

_BASE_AMP = 0.08


def _anchor_starts(signal_len, num_segments, seg_length):
    max_start = max(signal_len - seg_length, 0)
    head = 0.1 * signal_len
    tail = max(0.0, 0.78 * signal_len)
    anchors = np.linspace(head, tail, num_segments)
    return np.clip(np.round(anchors), 0, max_start).astype(np.int64)


def _body(x_ref, pi_ref, pq_ref, sc_ref, o_ref, trig_ref, *,
          starts, seg_len, sig_len):
    S = sig_len
    pat_i = jnp.tanh(pi_ref[...])
    pat_q = jnp.tanh(pq_ref[...])
    trig_ref[...] = jnp.zeros((1, 2 * S), jnp.float32)
    for k, s in enumerate(starts):
        e = min(s + seg_len, S)
        L = e - s
        g = jnp.maximum(sc_ref[k], 0.0)
        for base, pat in ((s, pat_i), (S + s, pat_q)):
            trig_ref[:, pl.ds(base, L)] += g * pat[:, :L]
    xv = x_ref[...]
    ssq = jnp.sum(xv * xv, axis=1, keepdims=True)
    amp = _BASE_AMP * jnp.sqrt(ssq / (2.0 * S) + 1e-12)
    o_ref[...] = xv + amp * trig_ref[...]


def kernel(x, pattern_i, pattern_q, segment_scale):
    batch, ch, S = x.shape
    L = pattern_i.shape[0]
    nseg = segment_scale.shape[0]
    starts = tuple(int(v) for v in _anchor_starts(S, nseg, L))

    x2 = x.reshape(batch, ch * S)
    TB = 64
    body = functools.partial(_body, starts=starts, seg_len=L, sig_len=S)
    out = pl.pallas_call(
        body,
        grid=(batch // TB,),
        in_specs=[
            pl.BlockSpec((TB, ch * S), lambda i: (i, 0)),
            pl.BlockSpec((1, L), lambda i: (0, 0)),
            pl.BlockSpec((1, L), lambda i: (0, 0)),
            pl.BlockSpec(memory_space=pltpu.SMEM),
        ],
        out_specs=pl.BlockSpec((TB, ch * S), lambda i: (i, 0)),
        out_shape=jax.ShapeDtypeStruct((batch, ch * S), jnp.float32),
        scratch_shapes=[pltpu.VMEM((1, ch * S), jnp.float32)],
        compiler_params=pltpu.CompilerParams(
            dimension_semantics=("parallel",)),
    )(x2, pattern_i.reshape(1, L), pattern_q.reshape(1, L), segment_scale)
    return out.reshape(batch, ch, S)


# speedup vs baseline: 14.7841x; 1.0039x over previous
import functools

import jax
import jax.numpy as jnp
import numpy as np
from jax.experimental import pallas as pl
from jax.experimental.pallas import tpu as pltpu

_BASE_AMP = 0.08


def _anchor_starts(signal_len, num_segments, seg_length):
    max_start = max(signal_len - seg_length, 0)
    head = 0.1 * signal_len
    tail = max(0.0, 0.78 * signal_len)
    anchors = np.linspace(head, tail, num_segments)
    return np.clip(np.round(anchors), 0, max_start).astype(np.int64)


def _body(x_ref, pi_ref, pq_ref, sc_ref, o_ref, trig_ref, *,
          starts, seg_len, sig_len):
    S = sig_len
    pat_i = jnp.tanh(pi_ref[...])
    pat_q = jnp.tanh(pq_ref[...])
    trig_ref[...] = jnp.zeros((1, 2 * S), jnp.float32)
    for k, s in enumerate(starts):
        e = min(s + seg_len, S)
        L = e - s
        g = jnp.maximum(sc_ref[k], 0.0)
        for base, pat in ((s, pat_i), (S + s, pat_q)):
            trig_ref[:, pl.ds(base, L)] += g * pat[:, :L]
    xv = x_ref[...]
    ssq = jnp.sum(xv * xv, axis=1, keepdims=True)
    amp = _BASE_AMP * jnp.sqrt(ssq / (2.0 * S) + 1e-12)
    o_ref[...] = xv + amp * trig_ref[...]


def kernel(x, pattern_i, pattern_q, segment_scale):
    batch, ch, S = x.shape
    L = pattern_i.shape[0]
    nseg = segment_scale.shape[0]
    starts = tuple(int(v) for v in _anchor_starts(S, nseg, L))

    x2 = x.reshape(batch, ch * S)
    TB = 128
    body = functools.partial(_body, starts=starts, seg_len=L, sig_len=S)
    out = pl.pallas_call(
        body,
        grid=(batch // TB,),
        in_specs=[
            pl.BlockSpec((TB, ch * S), lambda i: (i, 0)),
            pl.BlockSpec((1, L), lambda i: (0, 0)),
            pl.BlockSpec((1, L), lambda i: (0, 0)),
            pl.BlockSpec(memory_space=pltpu.SMEM),
        ],
        out_specs=pl.BlockSpec((TB, ch * S), lambda i: (i, 0)),
        out_shape=jax.ShapeDtypeStruct((batch, ch * S), jnp.float32),
        scratch_shapes=[pltpu.VMEM((1, ch * S), jnp.float32)],
        compiler_params=pltpu.CompilerParams(
            dimension_semantics=("parallel",)),
    )(x2, pattern_i.reshape(1, L), pattern_q.reshape(1, L), segment_scale)
    return out.reshape(batch, ch, S)


# speedup vs baseline: 53.6169x; 3.6267x over previous
import functools

import jax
import jax.numpy as jnp
import numpy as np
from jax.experimental import pallas as pl
from jax.experimental.pallas import tpu as pltpu

_BASE_AMP = 0.08


def _anchor_starts(signal_len, num_segments, seg_length):
    max_start = max(signal_len - seg_length, 0)
    head = 0.1 * signal_len
    tail = max(0.0, 0.78 * signal_len)
    anchors = np.linspace(head, tail, num_segments)
    return np.clip(np.round(anchors), 0, max_start).astype(np.int64)


def _body(x_ref, pi_ref, pq_ref, sc_ref, o_ref, trig_ref, *,
          starts, seg_len, sig_len):
    S = sig_len
    pat_i = jnp.tanh(pi_ref[...])
    pat_q = jnp.tanh(pq_ref[...])
    trig_ref[...] = jnp.zeros(trig_ref.shape, jnp.float32)
    for k, s in enumerate(starts):
        e = min(s + seg_len, S)
        L = e - s
        g = jnp.maximum(sc_ref[k], 0.0)
        trig_ref[0:1, pl.ds(s, L)] += g * pat_i[:, :L]
        trig_ref[1:2, pl.ds(s, L)] += g * pat_q[:, :L]
    xv = x_ref[...]
    ssq = jnp.sum(xv * xv, axis=(1, 2), keepdims=True)
    amp = _BASE_AMP * jnp.sqrt(ssq / (2.0 * S) + 1e-12)
    o_ref[...] = xv + amp * trig_ref[...][None, :, :]


def kernel(x, pattern_i, pattern_q, segment_scale):
    batch, ch, S = x.shape
    L = pattern_i.shape[0]
    nseg = segment_scale.shape[0]
    starts = tuple(int(v) for v in _anchor_starts(S, nseg, L))

    TB = 64
    body = functools.partial(_body, starts=starts, seg_len=L, sig_len=S)
    out = pl.pallas_call(
        body,
        grid=(batch // TB,),
        in_specs=[
            pl.BlockSpec((TB, ch, S), lambda i: (i, 0, 0)),
            pl.BlockSpec((1, L), lambda i: (0, 0)),
            pl.BlockSpec((1, L), lambda i: (0, 0)),
            pl.BlockSpec(memory_space=pltpu.SMEM),
        ],
        out_specs=pl.BlockSpec((TB, ch, S), lambda i: (i, 0, 0)),
        out_shape=jax.ShapeDtypeStruct((batch, ch, S), jnp.float32),
        scratch_shapes=[pltpu.VMEM((ch, S), jnp.float32)],
        compiler_params=pltpu.CompilerParams(
            dimension_semantics=("parallel",)),
    )(x, pattern_i.reshape(1, L), pattern_q.reshape(1, L), segment_scale)
    return out


# speedup vs baseline: 55.9419x; 1.0434x over previous
import functools

import jax
import jax.numpy as jnp
import numpy as np
from jax.experimental import pallas as pl
from jax.experimental.pallas import tpu as pltpu

_BASE_AMP = 0.08


def _anchor_starts(signal_len, num_segments, seg_length):
    max_start = max(signal_len - seg_length, 0)
    head = 0.1 * signal_len
    tail = max(0.0, 0.78 * signal_len)
    anchors = np.linspace(head, tail, num_segments)
    return np.clip(np.round(anchors), 0, max_start).astype(np.int64)


def _body(x_ref, pi_ref, pq_ref, sc_ref, o_ref, trig_ref, *,
          starts, seg_len, sig_len):
    S = sig_len
    pat_i = jnp.tanh(pi_ref[...])
    pat_q = jnp.tanh(pq_ref[...])
    trig_ref[...] = jnp.zeros(trig_ref.shape, jnp.float32)
    for k, s in enumerate(starts):
        e = min(s + seg_len, S)
        L = e - s
        g = jnp.maximum(sc_ref[k], 0.0)
        trig_ref[0:1, pl.ds(s, L)] += g * pat_i[:, :L]
        trig_ref[1:2, pl.ds(s, L)] += g * pat_q[:, :L]
    xv = x_ref[...]
    ssq = jnp.sum(xv * xv, axis=(1, 2), keepdims=True)
    amp = _BASE_AMP * jnp.sqrt(ssq / (2.0 * S) + 1e-12)
    o_ref[...] = xv + amp * trig_ref[...][None, :, :]


def kernel(x, pattern_i, pattern_q, segment_scale):
    batch, ch, S = x.shape
    L = pattern_i.shape[0]
    nseg = segment_scale.shape[0]
    starts = tuple(int(v) for v in _anchor_starts(S, nseg, L))

    TB = 128
    body = functools.partial(_body, starts=starts, seg_len=L, sig_len=S)
    out = pl.pallas_call(
        body,
        grid=(batch // TB,),
        in_specs=[
            pl.BlockSpec((TB, ch, S), lambda i: (i, 0, 0)),
            pl.BlockSpec((1, L), lambda i: (0, 0)),
            pl.BlockSpec((1, L), lambda i: (0, 0)),
            pl.BlockSpec(memory_space=pltpu.SMEM),
        ],
        out_specs=pl.BlockSpec((TB, ch, S), lambda i: (i, 0, 0)),
        out_shape=jax.ShapeDtypeStruct((batch, ch, S), jnp.float32),
        scratch_shapes=[pltpu.VMEM((ch, S), jnp.float32)],
        compiler_params=pltpu.CompilerParams(
            dimension_semantics=("parallel",)),
    )(x, pattern_i.reshape(1, L), pattern_q.reshape(1, L), segment_scale)
    return out
